# baseline (speedup 1.0000x reference)
import functools

import jax
import jax.numpy as jnp
from jax import lax
from jax.experimental import pallas as pl
from jax.experimental.pallas import tpu as pltpu
from jax.experimental.pallas import tpu_sc as plsc

C = 19
B, H, W = 8, 512, 512
R = 8
WH = W // 2
NC, NS, L = 2, 16, 16
NW = NC * NS
ROWS_PW = B * H // NW
CPW = ROWS_PW // R
WPB = H // ROWS_PW

_mesh = plsc.VectorSubcoreMesh(core_axis_name="c", subcore_axis_name="s")


@functools.partial(
    pl.kernel,
    mesh=_mesh,
    out_type=jax.ShapeDtypeStruct((B, C, H, W), jnp.float32),
    compiler_params=pltpu.CompilerParams(needs_layout_passes=False),
    scratch_types=[
        pltpu.VMEM((R, W), jnp.int32),
        pltpu.VMEM((R, W), jnp.int32),
        pltpu.VMEM((C, R, WH), jnp.float32),
        pltpu.VMEM((C, R, WH), jnp.float32),
        pltpu.SemaphoreType.DMA,
        pltpu.SemaphoreType.DMA,
        pltpu.SemaphoreType.DMA,
        pltpu.SemaphoreType.DMA,
    ],
)
def _onehot_sc(x_hbm, out_hbm, xa, xb, ob0, ob1, sem0, sem1, semxa, semxb):
    wid = lax.axis_index("s") * NC + lax.axis_index("c")
    b = wid // WPB
    h_base = (wid % WPB) * ROWS_PW

    obufs = (ob0, ob1)
    sems = (sem0, sem1)

    iota = lax.iota(jnp.int32, L)
    onesv = jnp.full((L,), 1.0, jnp.float32)
    zerosv = jnp.zeros((L,), jnp.float32)

    @plsc.parallel_loop(0, C, unroll=2)
    def _zero_body(i):
        for rr in range(R):
            for u in range(WH // L):
                ob0[i, rr, pl.ds(u * L, L)] = zerosv
                ob1[i, rr, pl.ds(u * L, L)] = zerosv

    def _scatter(ob, xref, half, val):
        @plsc.parallel_loop(0, WH // L, unroll=4)
        def body(wg):
            colbase = wg * L
            col = colbase + iota
            for r in range(R):
                xv = xref[r, pl.ds(half * WH + colbase, L)]
                plsc.store_scatter(
                    ob, [xv, jnp.full((L,), r, jnp.int32), col], val
                )

    def _load_x_async(t, xref, semx):
        h = pl.multiple_of(h_base + t * R, R)
        pltpu.async_copy(x_hbm.at[b, pl.ds(h, R), :], xref, semx)

    def _wait_x(xref, semx):
        pltpu.make_async_copy(x_hbm.at[0, pl.ds(0, R), :], xref, semx).wait()

    def _fire(t, k):
        h = pl.multiple_of(h_base + t * R, R)
        pltpu.async_copy(
            obufs[k],
            out_hbm.at[b, :, pl.ds(h, R), pl.ds(k * WH, WH)],
            sems[k],
        )

    def _drain(k):
        pltpu.make_async_copy(
            out_hbm.at[0, :, pl.ds(0, R), pl.ds(0, WH)], obufs[k], sems[k]
        ).wait()

    def _chunk(t, xref, xold, clear, wait_fn=None):
        for k in range(2):
            if clear:
                _drain(k)
                _scatter(obufs[k], xold, k, zerosv)
            if k == 0 and wait_fn is not None:
                wait_fn()
            _scatter(obufs[k], xref, k, onesv)
            _fire(t, k)

    _load_x_async(0, xa, semxa)
    _load_x_async(1, xb, semxb)
    _wait_x(xa, semxa)
    _chunk(0, xa, xb, clear=False)

    def outer(i, _):
        _chunk(2 * i + 1, xb, xa, clear=True,
               wait_fn=lambda: _wait_x(xb, semxb))
        _load_x_async(2 * i + 2, xa, semxa)
        _chunk(2 * i + 2, xa, xb, clear=True,
               wait_fn=lambda: _wait_x(xa, semxa))
        _load_x_async(2 * i + 3, xb, semxb)
        return 0

    lax.fori_loop(0, (CPW - 2) // 2, outer, 0)

    _chunk(CPW - 1, xb, xa, clear=True, wait_fn=lambda: _wait_x(xb, semxb))
    _drain(0)
    _drain(1)


def kernel(X_in, ones):
    del ones
    return _onehot_sc(X_in.astype(jnp.int32))


# speedup vs baseline: 1.0030x; 1.0030x over previous
import functools

import jax
import jax.numpy as jnp
from jax import lax
from jax.experimental import pallas as pl
from jax.experimental.pallas import tpu as pltpu
from jax.experimental.pallas import tpu_sc as plsc

C = 19
B, H, W = 8, 512, 512
R = 8
WH = W // 2
NC, NS, L = 2, 16, 16
NW = NC * NS
ROWS_PW = B * H // NW
CPW = ROWS_PW // R
WPB = H // ROWS_PW

_mesh = plsc.VectorSubcoreMesh(core_axis_name="c", subcore_axis_name="s")


@functools.partial(
    pl.kernel,
    mesh=_mesh,
    out_type=jax.ShapeDtypeStruct((B, C, H, W), jnp.float32),
    compiler_params=pltpu.CompilerParams(needs_layout_passes=False),
    scratch_types=[
        pltpu.VMEM((R, W), jnp.int32),
        pltpu.VMEM((R, W), jnp.int32),
        pltpu.VMEM((C, R, WH), jnp.float32),
        pltpu.VMEM((C, R, WH), jnp.float32),
        pltpu.SemaphoreType.DMA,
        pltpu.SemaphoreType.DMA,
        pltpu.SemaphoreType.DMA,
        pltpu.SemaphoreType.DMA,
    ],
)
def _onehot_sc(x_hbm, out_hbm, xa, xb, ob0, ob1, sem0, sem1, semxa, semxb):
    wid = lax.axis_index("s") * NC + lax.axis_index("c")
    b = wid // WPB
    h_base = (wid % WPB) * ROWS_PW

    obufs = (ob0, ob1)
    sems = (sem0, sem1)

    iota = lax.iota(jnp.int32, L)
    onesv = jnp.full((L,), 1.0, jnp.float32)
    zerosv = jnp.zeros((L,), jnp.float32)

    @plsc.parallel_loop(0, C, unroll=2)
    def _zero_body(i):
        for rr in range(R):
            for u in range(WH // L):
                ob0[i, rr, pl.ds(u * L, L)] = zerosv
                ob1[i, rr, pl.ds(u * L, L)] = zerosv

    def _scatter(ob, xref, half, val):
        @plsc.parallel_loop(0, WH // L, unroll=2)
        def body(wg):
            colbase = wg * L
            col = colbase + iota
            for r in range(R):
                xv = xref[r, pl.ds(half * WH + colbase, L)]
                plsc.store_scatter(
                    ob, [xv, jnp.full((L,), r, jnp.int32), col], val
                )

    def _scatter2(ob, xo, xn, half):
        @plsc.parallel_loop(0, WH // L, unroll=2)
        def body(wg):
            colbase = wg * L
            col = colbase + iota
            for r in range(R):
                rfull = jnp.full((L,), r, jnp.int32)
                xvo = xo[r, pl.ds(half * WH + colbase, L)]
                plsc.store_scatter(ob, [xvo, rfull, col], zerosv)
                xvn = xn[r, pl.ds(half * WH + colbase, L)]
                plsc.store_scatter(ob, [xvn, rfull, col], onesv)

    def _load_x_async(t, xref, semx):
        h = pl.multiple_of(h_base + t * R, R)
        pltpu.async_copy(x_hbm.at[b, pl.ds(h, R), :], xref, semx)

    def _wait_x(xref, semx):
        pltpu.make_async_copy(x_hbm.at[0, pl.ds(0, R), :], xref, semx).wait()

    def _fire(t, k):
        h = pl.multiple_of(h_base + t * R, R)
        pltpu.async_copy(
            obufs[k],
            out_hbm.at[b, :, pl.ds(h, R), pl.ds(k * WH, WH)],
            sems[k],
        )

    def _drain(k):
        pltpu.make_async_copy(
            out_hbm.at[0, :, pl.ds(0, R), pl.ds(0, WH)], obufs[k], sems[k]
        ).wait()

    def _chunk(t, xref, xold, clear, wait_fn=None):
        for k in range(2):
            if clear:
                _drain(k)
            if k == 0 and wait_fn is not None:
                wait_fn()
            if clear:
                _scatter2(obufs[k], xold, xref, k)
            else:
                _scatter(obufs[k], xref, k, onesv)
            _fire(t, k)

    _load_x_async(0, xa, semxa)
    _load_x_async(1, xb, semxb)
    _wait_x(xa, semxa)
    _chunk(0, xa, xb, clear=False)

    def outer(i, _):
        _chunk(2 * i + 1, xb, xa, clear=True,
               wait_fn=lambda: _wait_x(xb, semxb))
        _load_x_async(2 * i + 2, xa, semxa)
        _chunk(2 * i + 2, xa, xb, clear=True,
               wait_fn=lambda: _wait_x(xa, semxa))
        _load_x_async(2 * i + 3, xb, semxb)
        return 0

    lax.fori_loop(0, (CPW - 2) // 2, outer, 0)

    _chunk(CPW - 1, xb, xa, clear=True, wait_fn=lambda: _wait_x(xb, semxb))
    _drain(0)
    _drain(1)


def kernel(X_in, ones):
    del ones
    return _onehot_sc(X_in.astype(jnp.int32))


# speedup vs baseline: 1.0044x; 1.0014x over previous
import functools

import jax
import jax.numpy as jnp
from jax import lax
from jax.experimental import pallas as pl
from jax.experimental.pallas import tpu as pltpu
from jax.experimental.pallas import tpu_sc as plsc

C = 19
B, H, W = 8, 512, 512
R = 8
WH = W // 2
NC, NS, L = 2, 16, 16
NW = NC * NS
ROWS_PW = B * H // NW
CPW = ROWS_PW // R
WPB = H // ROWS_PW

_mesh = plsc.VectorSubcoreMesh(core_axis_name="c", subcore_axis_name="s")


@functools.partial(
    pl.kernel,
    mesh=_mesh,
    out_type=jax.ShapeDtypeStruct((B, C, H, W), jnp.float32),
    compiler_params=pltpu.CompilerParams(needs_layout_passes=False),
    scratch_types=[
        pltpu.VMEM((R, W), jnp.int32),
        pltpu.VMEM((R, W), jnp.int32),
        pltpu.VMEM((C, R, WH), jnp.float32),
        pltpu.VMEM((C, R, WH), jnp.float32),
        pltpu.SemaphoreType.DMA,
        pltpu.SemaphoreType.DMA,
        pltpu.SemaphoreType.DMA,
        pltpu.SemaphoreType.DMA,
    ],
)
def _onehot_sc(x_hbm, out_hbm, xa, xb, ob0, ob1, sem0, sem1, semxa, semxb):
    wid = lax.axis_index("s") * NC + lax.axis_index("c")
    b = wid // WPB
    h_base = (wid % WPB) * ROWS_PW

    obufs = (ob0, ob1)
    sems = (sem0, sem1)

    iota = lax.iota(jnp.int32, L)
    onesv = jnp.full((L,), 1.0, jnp.float32)
    zerosv = jnp.zeros((L,), jnp.float32)

    @plsc.parallel_loop(0, C, unroll=2)
    def _zero_body(i):
        for rr in range(R):
            for u in range(WH // L):
                ob0[i, rr, pl.ds(u * L, L)] = zerosv
                ob1[i, rr, pl.ds(u * L, L)] = zerosv

    def _scatter(ob, xref, half, val):
        @plsc.parallel_loop(0, WH // L, unroll=2)
        def body(wg):
            colbase = wg * L
            col = colbase + iota
            for r in range(R):
                xv = xref[r, pl.ds(half * WH + colbase, L)]
                plsc.store_scatter(
                    ob, [xv, jnp.full((L,), r, jnp.int32), col], val
                )

    def _load_x_async(t, xref, semx):
        h = pl.multiple_of(h_base + t * R, R)
        pltpu.async_copy(x_hbm.at[b, pl.ds(h, R), :], xref, semx)

    def _wait_x(xref, semx):
        pltpu.make_async_copy(x_hbm.at[0, pl.ds(0, R), :], xref, semx).wait()

    def _fire(t, k):
        h = pl.multiple_of(h_base + t * R, R)
        pltpu.async_copy(
            obufs[k],
            out_hbm.at[b, :, pl.ds(h, R), pl.ds(k * WH, WH)],
            sems[k],
        )

    def _drain(k):
        pltpu.make_async_copy(
            out_hbm.at[0, :, pl.ds(0, R), pl.ds(0, WH)], obufs[k], sems[k]
        ).wait()

    def _chunk(t, xref, xold, clear, wait_fn=None):
        for k in range(2):
            if clear:
                _drain(k)
                _scatter(obufs[k], xold, k, zerosv)
            if k == 0 and wait_fn is not None:
                wait_fn()
            _scatter(obufs[k], xref, k, onesv)
            _fire(t, k)

    _load_x_async(0, xa, semxa)
    _load_x_async(1, xb, semxb)
    _wait_x(xa, semxa)
    _chunk(0, xa, xb, clear=False)

    def outer(i, _):
        _chunk(2 * i + 1, xb, xa, clear=True,
               wait_fn=lambda: _wait_x(xb, semxb))
        _load_x_async(2 * i + 2, xa, semxa)
        _chunk(2 * i + 2, xa, xb, clear=True,
               wait_fn=lambda: _wait_x(xa, semxa))
        _load_x_async(2 * i + 3, xb, semxb)
        return 0

    lax.fori_loop(0, (CPW - 2) // 2, outer, 0)

    _chunk(CPW - 1, xb, xa, clear=True, wait_fn=lambda: _wait_x(xb, semxb))
    _drain(0)
    _drain(1)


def kernel(X_in, ones):
    del ones
    return _onehot_sc(X_in.astype(jnp.int32))


# speedup vs baseline: 1.0138x; 1.0094x over previous
import functools

import jax
import jax.numpy as jnp
from jax import lax
from jax.experimental import pallas as pl
from jax.experimental.pallas import tpu as pltpu
from jax.experimental.pallas import tpu_sc as plsc

C = 19
B, H, W = 8, 512, 512
R = 8
WH = W // 2
NC, NS, L = 2, 16, 16
NW = NC * NS
ROWS_PW = B * H // NW
CPW = ROWS_PW // R
WPB = H // ROWS_PW

_mesh = plsc.VectorSubcoreMesh(core_axis_name="c", subcore_axis_name="s")


@functools.partial(
    pl.kernel,
    mesh=_mesh,
    out_type=jax.ShapeDtypeStruct((B, C, H, W), jnp.float32),
    compiler_params=pltpu.CompilerParams(needs_layout_passes=False),
    scratch_types=[
        pltpu.VMEM((R, W), jnp.int32),
        pltpu.VMEM((R, W), jnp.int32),
        pltpu.VMEM((C, R, WH), jnp.float32),
        pltpu.VMEM((C, R, WH), jnp.float32),
        pltpu.SemaphoreType.DMA,
        pltpu.SemaphoreType.DMA,
        pltpu.SemaphoreType.DMA,
        pltpu.SemaphoreType.DMA,
    ],
)
def _onehot_sc(x_hbm, out_hbm, xa, xb, ob0, ob1, sem0, sem1, semxa, semxb):
    wid = lax.axis_index("s") * NC + lax.axis_index("c")
    b = wid // WPB
    h_base = (wid % WPB) * ROWS_PW

    obufs = (ob0, ob1)
    sems = (sem0, sem1)

    iota = lax.iota(jnp.int32, L)
    onesv = jnp.full((L,), 1.0, jnp.float32)
    zerosv = jnp.zeros((L,), jnp.float32)

    def _load_x_async(t, xref, semx):
        h = pl.multiple_of(h_base + t * R, R)
        pltpu.async_copy(x_hbm.at[b, pl.ds(h, R), :], xref, semx)

    _load_x_async(0, xa, semxa)
    _load_x_async(1, xb, semxb)

    @plsc.parallel_loop(0, C, unroll=2)
    def _zero_body(i):
        for rr in range(R):
            for u in range(WH // L):
                ob0[i, rr, pl.ds(u * L, L)] = zerosv
                ob1[i, rr, pl.ds(u * L, L)] = zerosv

    def _scatter(ob, xref, half, val):
        @plsc.parallel_loop(0, WH // L, unroll=2)
        def body(wg):
            colbase = wg * L
            col = colbase + iota
            for r in range(R):
                xv = xref[r, pl.ds(half * WH + colbase, L)]
                plsc.store_scatter(
                    ob, [xv, jnp.full((L,), r, jnp.int32), col], val
                )

    def _wait_x(xref, semx):
        pltpu.make_async_copy(x_hbm.at[0, pl.ds(0, R), :], xref, semx).wait()

    def _fire(t, k):
        h = pl.multiple_of(h_base + t * R, R)
        pltpu.async_copy(
            obufs[k],
            out_hbm.at[b, :, pl.ds(h, R), pl.ds(k * WH, WH)],
            sems[k],
        )

    def _drain(k):
        pltpu.make_async_copy(
            out_hbm.at[0, :, pl.ds(0, R), pl.ds(0, WH)], obufs[k], sems[k]
        ).wait()

    def _chunk(t, xref, xold, clear, wait_fn=None):
        for k in range(2):
            if clear:
                _drain(k)
                _scatter(obufs[k], xold, k, zerosv)
            if k == 0 and wait_fn is not None:
                wait_fn()
            _scatter(obufs[k], xref, k, onesv)
            _fire(t, k)

    _wait_x(xa, semxa)
    _chunk(0, xa, xb, clear=False)

    def outer(i, _):
        _chunk(2 * i + 1, xb, xa, clear=True,
               wait_fn=lambda: _wait_x(xb, semxb))
        _load_x_async(2 * i + 2, xa, semxa)
        _chunk(2 * i + 2, xa, xb, clear=True,
               wait_fn=lambda: _wait_x(xa, semxa))
        _load_x_async(2 * i + 3, xb, semxb)
        return 0

    lax.fori_loop(0, (CPW - 2) // 2, outer, 0)

    _chunk(CPW - 1, xb, xa, clear=True, wait_fn=lambda: _wait_x(xb, semxb))
    _drain(0)
    _drain(1)


def kernel(X_in, ones):
    del ones
    return _onehot_sc(X_in.astype(jnp.int32))
